# baseline (speedup 1.0000x reference)
import functools

import jax
import jax.numpy as jnp
from jax import lax
from jax.experimental import pallas as pl
from jax.experimental.pallas import tpu as pltpu
from jax.experimental.pallas import tpu_sc as plsc

NC, NS, L = 2, 16, 16
NW = NC * NS
B, D = 4096, 128
BPW = B // NW
CH = 16
ROUNDS = BPW // CH
GROUPS = CH // L
NBUF = 2
UNROLL = 4


def _rsqrt_max1(s):
    s = jnp.maximum(s, jnp.float32(1.0))
    i = plsc.bitcast(s, jnp.int32)
    i = jnp.int32(0x5F3759DF) - (i >> 1)
    y = plsc.bitcast(i, jnp.float32)
    for _ in range(3):
        y = y * (jnp.float32(1.5) - jnp.float32(0.5) * s * y * y)
    return y


def _coeffs(ss_he, ss_te, ss_re, ss_rt, d_ht, d_tt, d_hr, d_tr):
    s_h = _rsqrt_max1(ss_he)
    s_t = _rsqrt_max1(ss_te)
    s_r = _rsqrt_max1(ss_re)
    a_h = s_h * d_ht
    a_t = s_t * d_tt
    nsq_h = s_h * s_h * ss_he + jnp.float32(2.0) * s_h * a_h * d_hr + a_h * a_h * ss_rt
    nsq_t = s_t * s_t * ss_te + jnp.float32(2.0) * s_t * a_t * d_tr + a_t * a_t * ss_rt
    sg_h = _rsqrt_max1(nsq_h)
    sg_t = _rsqrt_max1(nsq_t)
    return sg_h * s_h, -(sg_t * s_t), s_r, sg_h * a_h - sg_t * a_t


def _group_hinge(pbufs, qbufs, row_idx):
    p_he, p_te, p_re, p_ht, p_tt, p_rt = pbufs
    q_he, q_te, q_re, q_ht, q_tt, q_rt = qbufs
    zeros = jnp.zeros((L,), jnp.float32)

    def pass1(d, acc):
        (pss_he, pss_te, pss_re, pss_rt, pd_ht, pd_tt, pd_hr, pd_tr,
         qss_he, qss_te, qss_re, qss_rt, qd_ht, qd_tt, qd_hr, qd_tr) = acc
        col = jnp.full((L,), d, jnp.int32)
        he = plsc.load_gather(p_he, [row_idx, col])
        te = plsc.load_gather(p_te, [row_idx, col])
        re = plsc.load_gather(p_re, [row_idx, col])
        ht = plsc.load_gather(p_ht, [row_idx, col])
        tt = plsc.load_gather(p_tt, [row_idx, col])
        rt = plsc.load_gather(p_rt, [row_idx, col])
        ge = plsc.load_gather(q_he, [row_idx, col])
        ue = plsc.load_gather(q_te, [row_idx, col])
        se = plsc.load_gather(q_re, [row_idx, col])
        gt = plsc.load_gather(q_ht, [row_idx, col])
        ut = plsc.load_gather(q_tt, [row_idx, col])
        st = plsc.load_gather(q_rt, [row_idx, col])
        return (pss_he + he * he, pss_te + te * te, pss_re + re * re,
                pss_rt + rt * rt, pd_ht + he * ht, pd_tt + te * tt,
                pd_hr + he * rt, pd_tr + te * rt,
                qss_he + ge * ge, qss_te + ue * ue, qss_re + se * se,
                qss_rt + st * st, qd_ht + ge * gt, qd_tt + ue * ut,
                qd_hr + ge * st, qd_tr + ue * st)

    acc = lax.fori_loop(0, D, pass1, (zeros,) * 16, unroll=UNROLL)
    pc1, pc2, pc3, pc4 = _coeffs(*acc[:8])
    qc1, qc2, qc3, qc4 = _coeffs(*acc[8:])

    def pass2(d, acc2):
        accp, accn = acc2
        col = jnp.full((L,), d, jnp.int32)
        he = plsc.load_gather(p_he, [row_idx, col])
        te = plsc.load_gather(p_te, [row_idx, col])
        re = plsc.load_gather(p_re, [row_idx, col])
        rt = plsc.load_gather(p_rt, [row_idx, col])
        ge = plsc.load_gather(q_he, [row_idx, col])
        ue = plsc.load_gather(q_te, [row_idx, col])
        se = plsc.load_gather(q_re, [row_idx, col])
        st = plsc.load_gather(q_rt, [row_idx, col])
        accp = accp + jnp.abs(pc1 * he + pc2 * te + pc3 * re + pc4 * rt)
        accn = accn + jnp.abs(qc1 * ge + qc2 * ue + qc3 * se + qc4 * st)
        return accp, accn

    p_score, n_score = lax.fori_loop(0, D, pass2, (zeros, zeros), unroll=UNROLL)
    return jnp.maximum(p_score - n_score + jnp.float32(1.0), jnp.float32(0.0))


def _make_kernel():
    mesh = plsc.VectorSubcoreMesh(
        core_axis_name="c", subcore_axis_name="s",
        num_cores=NC, num_subcores=NS)

    @functools.partial(
        pl.kernel, mesh=mesh,
        out_type=jax.ShapeDtypeStruct((NW, L), jnp.float32),
        compiler_params=pltpu.CompilerParams(needs_layout_passes=False),
        scratch_types=(
            [pltpu.VMEM((CH,), jnp.int32) for _ in range(6 * NBUF)]
            + [pltpu.VMEM((CH, D + 1), jnp.float32) for _ in range(12 * NBUF)]
            + [pltpu.VMEM((L,), jnp.float32)]
            + [pltpu.SemaphoreType.DMA for _ in range(NBUF)]
        ),
    )
    def trans_d(ph_h, pt_h, pr_h, nh_h, nt_h, nr_h,
                ent_e, rel_e, ent_t, rel_t, out, *scr):
        idxbufs = [scr[0:6], scr[6:12]]
        rowbufs = [scr[12:24], scr[24:36]]
        stage = scr[36]
        sems = scr[37:39]
        idx_hbm = (ph_h, pt_h, pr_h, nh_h, nt_h, nr_h)

        wid = lax.axis_index("s") * NC + lax.axis_index("c")
        iota = lax.iota(jnp.int32, L)

        def fire(r, bs):
            base = wid * BPW + r * CH
            for src, dst in zip(idx_hbm, idxbufs[bs]):
                pltpu.sync_copy(src.at[pl.ds(base, CH)], dst)
            i_ph, i_pt, i_pr, i_nh, i_nt, i_nr = idxbufs[bs]
            bufs = rowbufs[bs]
            pairs = ((ent_e, i_ph), (ent_e, i_pt), (rel_e, i_pr),
                     (ent_t, i_ph), (ent_t, i_pt), (rel_t, i_pr),
                     (ent_e, i_nh), (ent_e, i_nt), (rel_e, i_nr),
                     (ent_t, i_nh), (ent_t, i_nt), (rel_t, i_nr))
            return [pltpu.async_copy(tab.at[idx], dst.at[:, pl.ds(0, D)], sems[bs])
                    for (tab, idx), dst in zip(pairs, bufs)]

        total = jnp.zeros((L,), jnp.float32)
        copies = fire(0, 0)
        for r in range(ROUNDS):
            cur = r % NBUF
            for cp in copies:
                cp.wait()
            if r + 1 < ROUNDS:
                copies = fire(r + 1, (r + 1) % NBUF)
            bufs = rowbufs[cur]
            for g in range(GROUPS):
                row_idx = iota + jnp.int32(g * L)
                total = total + _group_hinge(bufs[0:6], bufs[6:12], row_idx)

        stage[...] = total
        pltpu.sync_copy(stage, out.at[wid])

    return trans_d


_TRANS_D = _make_kernel()


def kernel(pos_h, pos_t, pos_r, neg_h, neg_t, neg_r,
           ent_embeddings, rel_embeddings, ent_transfer, rel_transfer):
    idx = lambda a: a.reshape(-1).astype(jnp.int32)
    partials = _TRANS_D(idx(pos_h), idx(pos_t), idx(pos_r),
                        idx(neg_h), idx(neg_t), idx(neg_r),
                        ent_embeddings, rel_embeddings,
                        ent_transfer, rel_transfer)
    return jnp.sum(partials)


# speedup vs baseline: 4.0133x; 4.0133x over previous
import functools

import jax
import jax.numpy as jnp
from jax import lax
from jax.experimental import pallas as pl
from jax.experimental.pallas import tpu as pltpu
from jax.experimental.pallas import tpu_sc as plsc

NC, NS, L = 2, 16, 16
NW = NC * NS
B, D = 4096, 128
NCHUNK = D // L
BPW = B // NW
CH = 16
ROUNDS = BPW // CH
NBUF = 2


def _rsqrt_max1(s):
    s = jnp.maximum(s, jnp.float32(1.0))
    i = plsc.bitcast(s, jnp.int32)
    i = jnp.int32(0x5F3759DF) - (i >> 1)
    y = plsc.bitcast(i, jnp.float32)
    for _ in range(3):
        y = y * (jnp.float32(1.5) - jnp.float32(0.5) * s * y * y)
    return y


def _coeffs(ss_he, ss_te, ss_re, ss_rt, d_ht, d_tt, d_hr, d_tr):
    s_h = _rsqrt_max1(ss_he)
    s_t = _rsqrt_max1(ss_te)
    s_r = _rsqrt_max1(ss_re)
    a_h = s_h * d_ht
    a_t = s_t * d_tt
    nsq_h = s_h * s_h * ss_he + jnp.float32(2.0) * s_h * a_h * d_hr + a_h * a_h * ss_rt
    nsq_t = s_t * s_t * ss_te + jnp.float32(2.0) * s_t * a_t * d_tr + a_t * a_t * ss_rt
    sg_h = _rsqrt_max1(nsq_h)
    sg_t = _rsqrt_max1(nsq_t)
    return sg_h * s_h, -(sg_t * s_t), s_r, sg_h * a_h - sg_t * a_t


def _make_kernel():
    mesh = plsc.VectorSubcoreMesh(
        core_axis_name="c", subcore_axis_name="s",
        num_cores=NC, num_subcores=NS)

    @functools.partial(
        pl.kernel, mesh=mesh,
        out_type=jax.ShapeDtypeStruct((NW, L), jnp.float32),
        compiler_params=pltpu.CompilerParams(needs_layout_passes=False),
        scratch_types=(
            [pltpu.VMEM((BPW,), jnp.int32) for _ in range(6)]
            + [pltpu.VMEM((CH, D), jnp.float32) for _ in range(12 * NBUF)]
            + [pltpu.VMEM((16, 2 * L - 1), jnp.float32),
               pltpu.VMEM((8, 2 * L - 1), jnp.float32),
               pltpu.VMEM((2 * L - 1,), jnp.float32),
               pltpu.VMEM((L,), jnp.float32)]
            + [pltpu.SemaphoreType.DMA for _ in range(NBUF)]
        ),
    )
    def trans_d(ph_h, pt_h, pr_h, nh_h, nt_h, nr_h,
                ent_e, rel_e, ent_t, rel_t, out, *scr):
        ibufs = scr[0:6]
        rowbufs = [scr[6:18], scr[18:30]]
        sums_b, coef_b, hinge_b, stage = scr[30:34]
        sems = scr[34:36]
        idx_hbm = (ph_h, pt_h, pr_h, nh_h, nt_h, nr_h)

        wid = lax.axis_index("s") * NC + lax.axis_index("c")
        lane15 = lax.iota(jnp.int32, L) == jnp.int32(L - 1)
        base = wid * BPW
        for src, dst in zip(idx_hbm, ibufs):
            pltpu.sync_copy(src.at[pl.ds(base, BPW)], dst)
        i_ph, i_pt, i_pr, i_nh, i_nt, i_nr = ibufs

        def fire(r, bs):
            lo = r * CH
            bufs = rowbufs[bs]
            pairs = ((ent_e, i_ph), (ent_e, i_pt), (rel_e, i_pr),
                     (ent_t, i_ph), (ent_t, i_pt), (rel_t, i_pr),
                     (ent_e, i_nh), (ent_e, i_nt), (rel_e, i_nr),
                     (ent_t, i_nh), (ent_t, i_nt), (rel_t, i_nr))
            return [pltpu.async_copy(tab.at[idx.at[pl.ds(lo, CH)]], dst, sems[bs])
                    for (tab, idx), dst in zip(pairs, bufs)]

        total = jnp.zeros((L,), jnp.float32)
        copies = fire(0, 0)
        for r in range(ROUNDS):
            cur = r % NBUF
            for cp in copies:
                cp.wait()
            if r + 1 < ROUNDS:
                copies = fire(r + 1, (r + 1) % NBUF)
            (p_he, p_te, p_re, p_ht, p_tt, p_rt,
             q_he, q_te, q_re, q_ht, q_tt, q_rt) = rowbufs[cur]

            def pass1(i, _):
                z = jnp.zeros((L,), jnp.float32)
                acc = [z] * 16
                for c in range(NCHUNK):
                    sl = pl.ds(c * L, L)
                    he = p_he[i, sl]
                    te = p_te[i, sl]
                    re = p_re[i, sl]
                    ht = p_ht[i, sl]
                    tt = p_tt[i, sl]
                    rt = p_rt[i, sl]
                    ge = q_he[i, sl]
                    ue = q_te[i, sl]
                    se = q_re[i, sl]
                    gt = q_ht[i, sl]
                    ut = q_tt[i, sl]
                    st = q_rt[i, sl]
                    acc = [acc[0] + he * he, acc[1] + te * te,
                           acc[2] + re * re, acc[3] + rt * rt,
                           acc[4] + he * ht, acc[5] + te * tt,
                           acc[6] + he * rt, acc[7] + te * rt,
                           acc[8] + ge * ge, acc[9] + ue * ue,
                           acc[10] + se * se, acc[11] + st * st,
                           acc[12] + ge * gt, acc[13] + ue * ut,
                           acc[14] + ge * st, acc[15] + ue * st]
                for k in range(16):
                    plsc.store_compressed(sums_b.at[k, pl.ds(i, L)],
                                          plsc.cumsum(acc[k]), mask=lane15)
                return 0

            lax.fori_loop(0, CH, pass1, 0, unroll=2)

            srows = [sums_b[k, 0:L] for k in range(16)]
            pc1, pc2, pc3, pc4 = _coeffs(*srows[:8])
            qc1, qc2, qc3, qc4 = _coeffs(*srows[8:])
            for k, cv in enumerate((pc1, pc2, pc3, pc4, qc1, qc2, qc3, qc4)):
                coef_b[k, 0:L] = cv

            def pass2(i, _):
                cs = [jnp.full((L,), coef_b[k, pl.ds(i, L)][0], jnp.float32)
                      for k in range(8)]
                accp = jnp.zeros((L,), jnp.float32)
                accn = jnp.zeros((L,), jnp.float32)
                for c in range(NCHUNK):
                    sl = pl.ds(c * L, L)
                    vp = (cs[0] * p_he[i, sl] + cs[1] * p_te[i, sl]
                          + cs[2] * p_re[i, sl] + cs[3] * p_rt[i, sl])
                    vn = (cs[4] * q_he[i, sl] + cs[5] * q_te[i, sl]
                          + cs[6] * q_re[i, sl] + cs[7] * q_rt[i, sl])
                    accp = accp + jnp.abs(vp)
                    accn = accn + jnp.abs(vn)
                hinge = jnp.maximum(
                    plsc.cumsum(accp) - plsc.cumsum(accn) + jnp.float32(1.0),
                    jnp.float32(0.0))
                plsc.store_compressed(hinge_b.at[pl.ds(i, L)], hinge, mask=lane15)
                return 0

            lax.fori_loop(0, CH, pass2, 0, unroll=2)
            total = total + hinge_b[0:L]

        stage[...] = total
        pltpu.sync_copy(stage, out.at[wid])

    return trans_d


_TRANS_D = _make_kernel()


def kernel(pos_h, pos_t, pos_r, neg_h, neg_t, neg_r,
           ent_embeddings, rel_embeddings, ent_transfer, rel_transfer):
    idx = lambda a: a.reshape(-1).astype(jnp.int32)
    partials = _TRANS_D(idx(pos_h), idx(pos_t), idx(pos_r),
                        idx(neg_h), idx(neg_t), idx(neg_r),
                        ent_embeddings, rel_embeddings,
                        ent_transfer, rel_transfer)
    return jnp.sum(partials)
